# baseline (speedup 1.0000x reference)
import functools

import jax
import jax.numpy as jnp
from jax import lax
from jax.experimental import pallas as pl
from jax.experimental.pallas import tpu as pltpu
from jax.experimental.pallas import tpu_sc as plsc


_NC = 2
_NS = 16


def _node_dense_body(x_ref, w_ref, b_ref, o1, o2, o3, o4):
    acc = jnp.dot(x_ref[...], w_ref[...], preferred_element_type=jnp.float32)
    acc = acc + b_ref[...]
    d = o1.shape[1]
    o1[...] = acc[:, 0 * d:1 * d]
    o2[...] = acc[:, 1 * d:2 * d]
    o3[...] = acc[:, 2 * d:3 * d]
    o4[...] = acc[:, 3 * d:4 * d]


def _sigma_body(gsum_ref, ef_ref, w_ref, b_ref, s_ref):
    p = jnp.dot(ef_ref[...], w_ref[...], preferred_element_type=jnp.float32)
    s_ref[...] = jax.nn.sigmoid(gsum_ref[...] + p + b_ref[...])


def _edge_out_body(s_ref, ef_ref, sc_ref, bi_ref, eo_ref):
    y = s_ref[...] * ef_ref[...]
    mu = jnp.mean(y, axis=-1, keepdims=True)
    var = jnp.mean((y - mu) ** 2, axis=-1, keepdims=True)
    eo_ref[...] = (y - mu) * lax.rsqrt(var + 1e-5) * sc_ref[...] + bi_ref[...]


def _node_out_body(ah_ref, hn_ref, hd_ref, sc_ref, bi_ref, o_ref):
    x = ah_ref[...] + hn_ref[...] / (hd_ref[...] + 1e-8)
    mu = jnp.mean(x, axis=-1, keepdims=True)
    var = jnp.mean((x - mu) ** 2, axis=-1, keepdims=True)
    o_ref[...] = (x - mu) * lax.rsqrt(var + 1e-5) * sc_ref[...] + bi_ref[...]


def _make_sc_gather(E, D, C):
    NW = _NC * _NS
    EW = E // NW
    K = EW // C
    KB = (K - 1) // 2
    mesh = plsc.VectorSubcoreMesh(core_axis_name="c", subcore_axis_name="s",
                                  num_cores=_NC, num_subcores=_NS)

    buf_set = [
        pltpu.VMEM((C,), jnp.int32),
        pltpu.VMEM((C,), jnp.int32),
        pltpu.VMEM((C, D), jnp.float32),
        pltpu.VMEM((C, D), jnp.float32),
        pltpu.SemaphoreType.DMA,
        pltpu.SemaphoreType.DMA,
    ]

    @functools.partial(
        pl.kernel,
        out_type=jax.ShapeDtypeStruct((E, D), jnp.float32),
        mesh=mesh,
        scratch_types=buf_set + buf_set,
    )
    def k(src_hbm, dst_hbm, esrc_hbm, edst_hbm, gsum_hbm,
          si0, di0, ba0, bb0, sl0, ss0,
          si1, di1, ba1, bb1, sl1, ss1):
        wid = lax.axis_index("c") * _NS + lax.axis_index("s")
        sets = ((si0, di0, ba0, bb0, sl0, ss0),
                (si1, di1, ba1, bb1, sl1, ss1))

        def wait_store(sid):
            si, di, ba, bb, sl, ss = sets[sid]
            pltpu.make_async_copy(ba, gsum_hbm.at[pl.ds(0, C), :], ss).wait()

        def start(sid, ck):
            si, di, ba, bb, sl, ss = sets[sid]
            base = wid * EW + ck * C
            pltpu.sync_copy(src_hbm.at[pl.ds(base, C)], si)
            pltpu.sync_copy(dst_hbm.at[pl.ds(base, C)], di)
            pltpu.async_copy(esrc_hbm.at[si], ba, sl)
            pltpu.async_copy(edst_hbm.at[di], bb, sl)

        def finish(sid, ck):
            si, di, ba, bb, sl, ss = sets[sid]
            base = wid * EW + ck * C
            pltpu.make_async_copy(esrc_hbm.at[si], ba, sl).wait()
            pltpu.make_async_copy(edst_hbm.at[di], bb, sl).wait()

            def row(r, rc):
                for j in range(D // 16):
                    sl2 = pl.ds(j * 16, 16)
                    ba[r, sl2] = ba[r, sl2] + bb[r, sl2]
                return rc

            lax.fori_loop(0, C, row, 0)
            pltpu.async_copy(ba, gsum_hbm.at[pl.ds(base, C), :], ss)

        start(0, 0)

        def body(k2, carry):
            c0 = 2 * k2

            @pl.when(k2 > 0)
            def _():
                wait_store(1)

            start(1, c0 + 1)
            finish(0, c0)
            wait_store(0)
            start(0, c0 + 2)
            finish(1, c0 + 1)
            return carry

        lax.fori_loop(0, KB, body, 0)
        finish(0, K - 1)
        wait_store(1)
        wait_store(0)

    return k


def _make_sc_scatter(E, NP, D, C):
    ET = E // _NS
    K = ET // C
    KB = K // 2
    RPT = NP // _NS
    mesh = plsc.VectorSubcoreMesh(core_axis_name="c", subcore_axis_name="s",
                                  num_cores=_NC, num_subcores=_NS)

    buf_set = [
        pltpu.VMEM((C,), jnp.int32),
        pltpu.VMEM((C,), jnp.int32),
        pltpu.VMEM((C, D), jnp.float32),
        pltpu.VMEM((C, D), jnp.float32),
        pltpu.SemaphoreType.DMA,
        pltpu.SemaphoreType.DMA,
    ]

    @functools.partial(
        pl.kernel,
        out_type=(jax.ShapeDtypeStruct((NP, D), jnp.float32),
                  jax.ShapeDtypeStruct((NP, D), jnp.float32)),
        mesh=mesh,
        scratch_types=[pltpu.VMEM_SHARED((NP, D), jnp.float32)]
        + buf_set + buf_set,
    )
    def k(src_hbm, dst_hbm, s_hbm, bh_hbm, z_hbm, acch_hbm, accs_hbm,
          acc, di0, si0, bs0, bg0, sl0, sc0,
          di1, si1, bs1, bg1, sl1, sc1):
        c = lax.axis_index("c")
        s = lax.axis_index("s")
        r0 = s * RPT
        pltpu.sync_copy(z_hbm.at[pl.ds(r0, RPT), :], acc.at[pl.ds(r0, RPT), :])
        plsc.subcore_barrier()
        sets = ((di0, si0, bs0, bg0, sl0, sc0),
                (di1, si1, bs1, bg1, sl1, sc1))

        def start(sid, ck):
            di, si, bs, bg, sl, sc = sets[sid]
            base = s * ET + ck * C
            pltpu.sync_copy(dst_hbm.at[pl.ds(base, C)], di)
            pltpu.async_copy(s_hbm.at[pl.ds(base, C), :], bs, sl)

            @pl.when(c == 0)
            def _():
                pltpu.sync_copy(src_hbm.at[pl.ds(base, C)], si)
                pltpu.async_copy(bh_hbm.at[si], bg, sl)

        def finish(sid, ck):
            di, si, bs, bg, sl, sc = sets[sid]
            base = s * ET + ck * C
            pltpu.make_async_copy(s_hbm.at[pl.ds(base, C), :], bs, sl).wait()

            @pl.when(c == 0)
            def _():
                pltpu.make_async_copy(bh_hbm.at[si], bg, sl).wait()

                @plsc.parallel_loop(0, C, unroll=4)
                def _mul(r):
                    for j in range(D // 16):
                        sl2 = pl.ds(j * 16, 16)
                        bs[r, sl2] = bs[r, sl2] * bg[r, sl2]

            pltpu.async_copy(bs, acc.at[di], sc, add=True)

        def wait_scatter(sid):
            di, si, bs, bg, sl, sc = sets[sid]
            pltpu.make_async_copy(bs, acc.at[di], sc).wait()

        start(0, 0)

        def body(k2, carry):
            c0 = 2 * k2

            @pl.when(k2 > 0)
            def _():
                wait_scatter(1)

            start(1, c0 + 1)
            finish(0, c0)

            @pl.when(k2 < KB - 1)
            def _():
                wait_scatter(0)
                start(0, c0 + 2)

            finish(1, c0 + 1)
            return carry

        lax.fori_loop(0, KB, body, 0)
        wait_scatter(0)
        wait_scatter(1)
        plsc.subcore_barrier()

        @pl.when(c == 0)
        def _():
            pltpu.sync_copy(acc.at[pl.ds(r0, RPT), :],
                            acch_hbm.at[pl.ds(r0, RPT), :])

        @pl.when(c == 1)
        def _():
            pltpu.sync_copy(acc.at[pl.ds(r0, RPT), :],
                            accs_hbm.at[pl.ds(r0, RPT), :])

    return k


def kernel(node_feats, edge_feats, edge_index,
           W_src_gate, b_src_gate, W_dst_gate, b_dst_gate,
           W_edge_gate, b_edge_gate, W_src_update, b_src_update,
           W_dst_update, b_dst_update,
           ln_e_scale, ln_e_bias, ln_n_scale, ln_n_bias):
    N, D = node_feats.shape
    E = edge_feats.shape[0]
    BN = 2000
    BE = 4000
    C = 80

    src = edge_index[0]
    dst = edge_index[1]

    Wn = jnp.concatenate([W_src_gate, W_dst_gate, W_dst_update, W_src_update],
                         axis=1)
    bn = jnp.concatenate([b_src_gate, b_dst_gate, b_dst_update, b_src_update]
                         ).reshape(1, 4 * D)
    esrc, edst, bh, ah = pl.pallas_call(
        _node_dense_body,
        grid=(N // BN,),
        in_specs=[
            pl.BlockSpec((BN, D), lambda i: (i, 0)),
            pl.BlockSpec((D, 4 * D), lambda i: (0, 0)),
            pl.BlockSpec((1, 4 * D), lambda i: (0, 0)),
        ],
        out_specs=[pl.BlockSpec((BN, D), lambda i: (i, 0))] * 4,
        out_shape=[jax.ShapeDtypeStruct((N, D), jnp.float32)] * 4,
    )(node_feats, Wn, bn)

    gsum = _make_sc_gather(E, D, C)(src, dst, esrc, edst)

    sgm = pl.pallas_call(
        _sigma_body,
        grid=(E // BE,),
        in_specs=[
            pl.BlockSpec((BE, D), lambda i: (i, 0)),
            pl.BlockSpec((BE, D), lambda i: (i, 0)),
            pl.BlockSpec((D, D), lambda i: (0, 0)),
            pl.BlockSpec((1, D), lambda i: (0, 0)),
        ],
        out_specs=pl.BlockSpec((BE, D), lambda i: (i, 0)),
        out_shape=jax.ShapeDtypeStruct((E, D), jnp.float32),
    )(gsum, edge_feats, W_edge_gate, b_edge_gate.reshape(1, D))

    NP = ((N + 8 * _NS - 1) // (8 * _NS)) * (8 * _NS)
    zeros = jnp.zeros((NP, D), jnp.float32)
    acc_h, acc_s = _make_sc_scatter(E, NP, D, C)(src, dst, sgm, bh, zeros)
    acc_h = acc_h[:N]
    acc_s = acc_s[:N]

    edge_out = pl.pallas_call(
        _edge_out_body,
        grid=(E // BE,),
        in_specs=[
            pl.BlockSpec((BE, D), lambda i: (i, 0)),
            pl.BlockSpec((BE, D), lambda i: (i, 0)),
            pl.BlockSpec((1, D), lambda i: (0, 0)),
            pl.BlockSpec((1, D), lambda i: (0, 0)),
        ],
        out_specs=pl.BlockSpec((BE, D), lambda i: (i, 0)),
        out_shape=jax.ShapeDtypeStruct((E, D), jnp.float32),
    )(sgm, edge_feats, ln_e_scale.reshape(1, D), ln_e_bias.reshape(1, D))

    node_out = pl.pallas_call(
        _node_out_body,
        grid=(N // BN,),
        in_specs=[
            pl.BlockSpec((BN, D), lambda i: (i, 0)),
            pl.BlockSpec((BN, D), lambda i: (i, 0)),
            pl.BlockSpec((BN, D), lambda i: (i, 0)),
            pl.BlockSpec((1, D), lambda i: (0, 0)),
            pl.BlockSpec((1, D), lambda i: (0, 0)),
        ],
        out_specs=pl.BlockSpec((BN, D), lambda i: (i, 0)),
        out_shape=jax.ShapeDtypeStruct((N, D), jnp.float32),
    )(ah, acc_h, acc_s, ln_n_scale.reshape(1, D), ln_n_bias.reshape(1, D))

    return node_out, edge_out


# speedup vs baseline: 1.0757x; 1.0757x over previous
import functools

import jax
import jax.numpy as jnp
from jax import lax
from jax.experimental import pallas as pl
from jax.experimental.pallas import tpu as pltpu
from jax.experimental.pallas import tpu_sc as plsc


_NC = 2
_NS = 16


def _node_dense_body(x_ref, w_ref, b_ref, o1, o2, o3, o4):
    acc = jnp.dot(x_ref[...], w_ref[...], preferred_element_type=jnp.float32)
    acc = acc + b_ref[...]
    d = o1.shape[1]
    o1[...] = acc[:, 0 * d:1 * d]
    o2[...] = acc[:, 1 * d:2 * d]
    o3[...] = acc[:, 2 * d:3 * d]
    o4[...] = acc[:, 3 * d:4 * d]


def _edgewise_body(gsum_ref, ef_ref, w_ref, b_ref, sc_ref, bi_ref,
                   eo_ref, s_ref):
    p = jnp.dot(ef_ref[...], w_ref[...], preferred_element_type=jnp.float32)
    sg = jax.nn.sigmoid(gsum_ref[...].astype(jnp.float32) + p + b_ref[...])
    y = sg * ef_ref[...]
    mu = jnp.mean(y, axis=-1, keepdims=True)
    var = jnp.mean((y - mu) ** 2, axis=-1, keepdims=True)
    eo_ref[...] = (y - mu) * lax.rsqrt(var + 1e-5) * sc_ref[...] + bi_ref[...]
    s_ref[...] = sg


def _node_out_body(ah_ref, hn_ref, hd_ref, sc_ref, bi_ref, o_ref):
    x = ah_ref[...] + hn_ref[...] / (hd_ref[...] + 1e-8)
    mu = jnp.mean(x, axis=-1, keepdims=True)
    var = jnp.mean((x - mu) ** 2, axis=-1, keepdims=True)
    o_ref[...] = (x - mu) * lax.rsqrt(var + 1e-5) * sc_ref[...] + bi_ref[...]


def _make_sc_gather(E, D, C):
    NW = _NC * _NS
    EW = E // NW
    K = EW // C
    KB = (K - 1) // 2
    mesh = plsc.VectorSubcoreMesh(core_axis_name="c", subcore_axis_name="s",
                                  num_cores=_NC, num_subcores=_NS)

    buf_set = [
        pltpu.VMEM((C,), jnp.int32),
        pltpu.VMEM((C,), jnp.int32),
        pltpu.VMEM((C, D), jnp.float32),
        pltpu.VMEM((C, D), jnp.float32),
        pltpu.SemaphoreType.DMA,
        pltpu.SemaphoreType.DMA,
    ]

    @functools.partial(
        pl.kernel,
        out_type=jax.ShapeDtypeStruct((E, D), jnp.float32),
        mesh=mesh,
        scratch_types=buf_set + buf_set,
    )
    def k(src_hbm, dst_hbm, esrc_hbm, edst_hbm, gsum_hbm,
          si0, di0, ba0, bb0, sl0, ss0,
          si1, di1, ba1, bb1, sl1, ss1):
        wid = lax.axis_index("c") * _NS + lax.axis_index("s")
        sets = ((si0, di0, ba0, bb0, sl0, ss0),
                (si1, di1, ba1, bb1, sl1, ss1))

        def wait_store(sid):
            si, di, ba, bb, sl, ss = sets[sid]
            pltpu.make_async_copy(ba, gsum_hbm.at[pl.ds(0, C), :], ss).wait()

        def start(sid, ck):
            si, di, ba, bb, sl, ss = sets[sid]
            base = wid * EW + ck * C
            pltpu.sync_copy(src_hbm.at[pl.ds(base, C)], si)
            pltpu.sync_copy(dst_hbm.at[pl.ds(base, C)], di)
            pltpu.async_copy(esrc_hbm.at[si], ba, sl)
            pltpu.async_copy(edst_hbm.at[di], bb, sl)

        def finish(sid, ck):
            si, di, ba, bb, sl, ss = sets[sid]
            base = wid * EW + ck * C
            pltpu.make_async_copy(esrc_hbm.at[si], ba, sl).wait()
            pltpu.make_async_copy(edst_hbm.at[di], bb, sl).wait()

            @plsc.parallel_loop(0, C, unroll=4)
            def _add(r):
                for j in range(D // 16):
                    sl2 = pl.ds(j * 16, 16)
                    ba[r, sl2] = ba[r, sl2] + bb[r, sl2]

            pltpu.async_copy(ba, gsum_hbm.at[pl.ds(base, C), :], ss)

        start(0, 0)

        def body(k2, carry):
            c0 = 2 * k2

            @pl.when(k2 > 0)
            def _():
                wait_store(1)

            start(1, c0 + 1)
            finish(0, c0)
            wait_store(0)
            start(0, c0 + 2)
            finish(1, c0 + 1)
            return carry

        lax.fori_loop(0, KB, body, 0)
        finish(0, K - 1)
        wait_store(1)
        wait_store(0)

    return k


def _make_sc_scatter(E, NP, D, C):
    ET = E // _NS
    K = ET // C
    KB = K // 2
    RPT = NP // _NS
    mesh = plsc.VectorSubcoreMesh(core_axis_name="c", subcore_axis_name="s",
                                  num_cores=_NC, num_subcores=_NS)

    buf_set = [
        pltpu.VMEM((C,), jnp.int32),
        pltpu.VMEM((C,), jnp.int32),
        pltpu.VMEM((C, D), jnp.float32),
        pltpu.VMEM((C, D), jnp.float32),
        pltpu.SemaphoreType.DMA,
        pltpu.SemaphoreType.DMA,
    ]

    @functools.partial(
        pl.kernel,
        out_type=(jax.ShapeDtypeStruct((NP, D), jnp.float32),
                  jax.ShapeDtypeStruct((NP, D), jnp.float32)),
        mesh=mesh,
        scratch_types=[pltpu.VMEM_SHARED((NP, D), jnp.float32)]
        + buf_set + buf_set,
    )
    def k(src_hbm, dst_hbm, s_hbm, bh_hbm, z_hbm, acch_hbm, accs_hbm,
          acc, di0, si0, bs0, bg0, sl0, sc0,
          di1, si1, bs1, bg1, sl1, sc1):
        c = lax.axis_index("c")
        s = lax.axis_index("s")
        r0 = s * RPT
        pltpu.sync_copy(z_hbm.at[pl.ds(r0, RPT), :], acc.at[pl.ds(r0, RPT), :])
        plsc.subcore_barrier()
        sets = ((di0, si0, bs0, bg0, sl0, sc0),
                (di1, si1, bs1, bg1, sl1, sc1))

        def start(sid, ck):
            di, si, bs, bg, sl, sc = sets[sid]
            base = s * ET + ck * C
            pltpu.sync_copy(dst_hbm.at[pl.ds(base, C)], di)
            pltpu.async_copy(s_hbm.at[pl.ds(base, C), :], bs, sl)

            @pl.when(c == 0)
            def _():
                pltpu.sync_copy(src_hbm.at[pl.ds(base, C)], si)
                pltpu.async_copy(bh_hbm.at[si], bg, sl)

        def finish(sid, ck):
            di, si, bs, bg, sl, sc = sets[sid]
            base = s * ET + ck * C
            pltpu.make_async_copy(s_hbm.at[pl.ds(base, C), :], bs, sl).wait()

            @pl.when(c == 0)
            def _():
                pltpu.make_async_copy(bh_hbm.at[si], bg, sl).wait()

                @plsc.parallel_loop(0, C, unroll=4)
                def _mul(r):
                    for j in range(D // 16):
                        sl2 = pl.ds(j * 16, 16)
                        bs[r, sl2] = bs[r, sl2] * bg[r, sl2]

            pltpu.async_copy(bs, acc.at[di], sc, add=True)

        def wait_scatter(sid):
            di, si, bs, bg, sl, sc = sets[sid]
            pltpu.make_async_copy(bs, acc.at[di], sc).wait()

        start(0, 0)

        def body(k2, carry):
            c0 = 2 * k2

            @pl.when(k2 > 0)
            def _():
                wait_scatter(1)

            start(1, c0 + 1)
            finish(0, c0)

            @pl.when(k2 < KB - 1)
            def _():
                wait_scatter(0)
                start(0, c0 + 2)

            finish(1, c0 + 1)
            return carry

        lax.fori_loop(0, KB, body, 0)
        wait_scatter(0)
        wait_scatter(1)
        plsc.subcore_barrier()

        @pl.when(c == 0)
        def _():
            pltpu.sync_copy(acc.at[pl.ds(r0, RPT), :],
                            acch_hbm.at[pl.ds(r0, RPT), :])

        @pl.when(c == 1)
        def _():
            pltpu.sync_copy(acc.at[pl.ds(r0, RPT), :],
                            accs_hbm.at[pl.ds(r0, RPT), :])

    return k


def kernel(node_feats, edge_feats, edge_index,
           W_src_gate, b_src_gate, W_dst_gate, b_dst_gate,
           W_edge_gate, b_edge_gate, W_src_update, b_src_update,
           W_dst_update, b_dst_update,
           ln_e_scale, ln_e_bias, ln_n_scale, ln_n_bias):
    N, D = node_feats.shape
    E = edge_feats.shape[0]
    BN = 2000
    BE = 4000
    C = 80

    src = edge_index[0]
    dst = edge_index[1]

    Wn = jnp.concatenate([W_src_gate, W_dst_gate, W_dst_update, W_src_update],
                         axis=1)
    bn = jnp.concatenate([b_src_gate, b_dst_gate, b_dst_update, b_src_update]
                         ).reshape(1, 4 * D)
    esrc, edst, bh, ah = pl.pallas_call(
        _node_dense_body,
        grid=(N // BN,),
        in_specs=[
            pl.BlockSpec((BN, D), lambda i: (i, 0)),
            pl.BlockSpec((D, 4 * D), lambda i: (0, 0)),
            pl.BlockSpec((1, 4 * D), lambda i: (0, 0)),
        ],
        out_specs=[pl.BlockSpec((BN, D), lambda i: (i, 0))] * 4,
        out_shape=[jax.ShapeDtypeStruct((N, D), jnp.float32)] * 4,
    )(node_feats, Wn, bn)

    gsum = _make_sc_gather(E, D, C)(src, dst, esrc, edst)

    edge_out, sgm = pl.pallas_call(
        _edgewise_body,
        grid=(E // BE,),
        in_specs=[
            pl.BlockSpec((BE, D), lambda i: (i, 0)),
            pl.BlockSpec((BE, D), lambda i: (i, 0)),
            pl.BlockSpec((D, D), lambda i: (0, 0)),
            pl.BlockSpec((1, D), lambda i: (0, 0)),
            pl.BlockSpec((1, D), lambda i: (0, 0)),
            pl.BlockSpec((1, D), lambda i: (0, 0)),
        ],
        out_specs=[pl.BlockSpec((BE, D), lambda i: (i, 0))] * 2,
        out_shape=[jax.ShapeDtypeStruct((E, D), jnp.float32)] * 2,
    )(gsum, edge_feats, W_edge_gate, b_edge_gate.reshape(1, D),
      ln_e_scale.reshape(1, D), ln_e_bias.reshape(1, D))

    NP = ((N + 8 * _NS - 1) // (8 * _NS)) * (8 * _NS)
    zeros = jnp.zeros((NP, D), jnp.float32)
    acc_h, acc_s = _make_sc_scatter(E, NP, D, C)(src, dst, sgm, bh, zeros)
    acc_h = acc_h[:N]
    acc_s = acc_s[:N]

    node_out = pl.pallas_call(
        _node_out_body,
        grid=(N // BN,),
        in_specs=[
            pl.BlockSpec((BN, D), lambda i: (i, 0)),
            pl.BlockSpec((BN, D), lambda i: (i, 0)),
            pl.BlockSpec((BN, D), lambda i: (i, 0)),
            pl.BlockSpec((1, D), lambda i: (0, 0)),
            pl.BlockSpec((1, D), lambda i: (0, 0)),
        ],
        out_specs=pl.BlockSpec((BN, D), lambda i: (i, 0)),
        out_shape=jax.ShapeDtypeStruct((N, D), jnp.float32),
    )(ah, acc_h, acc_s, ln_n_scale.reshape(1, D), ln_n_bias.reshape(1, D))

    return node_out, edge_out
